# Initial kernel scaffold; baseline (speedup 1.0000x reference)
import jax
import jax.numpy as jnp
from jax.experimental import pallas as pl


def kernel(x, r_w1, r_bn1_w, r_bn1_b, r_bn1_m, r_bn1_v, r_w2, r_bn2_w, r_bn2_b, r_bn2_m, r_bn2_v, s_w, s_bn_w, s_bn_b, s_bn_m, s_bn_v, e_w1, e_bn1_w, e_bn1_b, e_bn1_m, e_bn1_v, e_w2, e_bn2_w, e_bn2_b, e_bn2_m, e_bn2_v):
    raise NotImplementedError("write your pallas kernel here")


# speedup vs baseline: 10.2698x; 10.2698x over previous
import functools

import jax
import jax.numpy as jnp
from jax.experimental import pallas as pl
from jax.experimental.pallas import tpu as pltpu

B = 4; CIN = 96; COUT = 96; H = 224; W = 224
E = 8; TOPK = 2; HID = 192; RED = 12; POOL = 4
HP = H // POOL
NPIX = H * W
NPP = HP * HP


def _silu(v):
    return v * jax.nn.sigmoid(v)


def _pool_kernel(x_ref, o_ref):
    cb = x_ref.shape[0]
    a = x_ref[...].reshape(cb * H, W)
    r = jax.lax.broadcasted_iota(jnp.int32, (W, HP), 0)
    c = jax.lax.broadcasted_iota(jnp.int32, (W, HP), 1)
    p = jnp.where(r // POOL == c, 1.0 / (POOL * POOL), 0.0)
    aw = jnp.dot(a, p, preferred_element_type=jnp.float32)
    a3 = aw.reshape(cb, H, HP)
    rt = jax.lax.broadcasted_iota(jnp.int32, (HP, H), 0)
    ct = jax.lax.broadcasted_iota(jnp.int32, (HP, H), 1)
    pt = jnp.where(ct // POOL == rt, 1.0, 0.0)
    for ch in range(cb):
        o_ref[ch] = jnp.dot(pt, a3[ch], preferred_element_type=jnp.float32)


def _pool(x):
    x4 = x.reshape(B * CIN, H, W)
    cb = 8
    return pl.pallas_call(
        _pool_kernel,
        grid=(B * CIN // cb,),
        in_specs=[pl.BlockSpec((cb, H, W), lambda i: (i, 0, 0))],
        out_specs=pl.BlockSpec((cb, HP, HP), lambda i: (i, 0, 0)),
        out_shape=jax.ShapeDtypeStruct((B * CIN, HP, HP), jnp.float32),
        compiler_params=pltpu.CompilerParams(
            dimension_semantics=("arbitrary",)),
    )(x4)


def _router_kernel(xd_ref, w1t_ref, c1_ref, w2_ref, c2_ref, ti_ref, tv_ref):
    xd = xd_ref[0]
    pad = HP + 1
    xpad = jnp.pad(xd, ((0, 0), (pad, pad)))
    jcol = jax.lax.broadcasted_iota(jnp.int32, (1, NPP), 1) % HP
    acc = jnp.zeros((RED, NPP), jnp.float32)
    t = 0
    for di in (-1, 0, 1):
        for dj in (-1, 0, 1):
            s = HP * di + dj
            xs = xpad[:, pad + s:pad + s + NPP]
            if dj == -1:
                xs = xs * (jcol >= 1).astype(jnp.float32)
            elif dj == 1:
                xs = xs * (jcol <= HP - 2).astype(jnp.float32)
            acc = acc + jnp.dot(w1t_ref[t], xs,
                                preferred_element_type=jnp.float32)
            t += 1
    h = _silu(acc + c1_ref[...])
    m = jnp.sum(h, axis=1, keepdims=True) * (1.0 / NPP)
    gl = jnp.dot(w2_ref[...], m, preferred_element_type=jnp.float32) + c2_ref[...]
    ex = jnp.exp(gl - jnp.max(gl))
    prob = ex / jnp.sum(ex)
    ie = jax.lax.broadcasted_iota(jnp.int32, (E, 1), 0)
    v1 = jnp.max(prob)
    i1 = jnp.min(jnp.where(prob == v1, ie, E))
    p2 = jnp.where(ie == i1, -1.0, prob)
    v2 = jnp.max(p2)
    i2 = jnp.min(jnp.where(p2 == v2, ie, E))
    ssum = v1 + v2 + 1e-6
    iv = jax.lax.broadcasted_iota(jnp.int32, (1, 1, TOPK), 2)
    ti_ref[...] = jnp.where(iv == 0, i1, i2).astype(jnp.int32)
    tv_ref[...] = jnp.where(iv == 0, v1 / ssum, v2 / ssum)


def _router(xdf, w1t, c1, w2, c2):
    return pl.pallas_call(
        _router_kernel,
        grid=(B,),
        in_specs=[
            pl.BlockSpec((1, CIN, NPP), lambda b: (b, 0, 0)),
            pl.BlockSpec((9, RED, CIN), lambda b: (0, 0, 0)),
            pl.BlockSpec((RED, 1), lambda b: (0, 0)),
            pl.BlockSpec((E, RED), lambda b: (0, 0)),
            pl.BlockSpec((E, 1), lambda b: (0, 0)),
        ],
        out_specs=[
            pl.BlockSpec((1, 1, TOPK), lambda b: (b, 0, 0)),
            pl.BlockSpec((1, 1, TOPK), lambda b: (b, 0, 0)),
        ],
        out_shape=[
            jax.ShapeDtypeStruct((B, 1, TOPK), jnp.int32),
            jax.ShapeDtypeStruct((B, 1, TOPK), jnp.float32),
        ],
        compiler_params=pltpu.CompilerParams(
            dimension_semantics=("arbitrary",)),
    )(xdf, w1t, c1, w2, c2)


def _main_kernel(idx_ref, x_ref, tv_ref, w1a_ref, w1b_ref, c1a_ref, c1b_ref,
                 w2a_ref, w2b_ref, c2a_ref, c2b_ref, ws_ref, cs_ref, o_ref):
    xt = x_ref[0]
    w0 = tv_ref[0, 0, 0]
    w1 = tv_ref[0, 0, 1]
    sh = _silu(jnp.dot(ws_ref[...], xt, preferred_element_type=jnp.float32)
               + cs_ref[...])
    ha = _silu(jnp.dot(w1a_ref[0], xt, preferred_element_type=jnp.float32)
               + c1a_ref[0])
    hb = _silu(jnp.dot(w1b_ref[0], xt, preferred_element_type=jnp.float32)
               + c1b_ref[0])
    oa = jnp.dot(w2a_ref[0], ha, preferred_element_type=jnp.float32)
    ob = jnp.dot(w2b_ref[0], hb, preferred_element_type=jnp.float32)
    cc = w0 * c2a_ref[0] + w1 * c2b_ref[0]
    o_ref[0] = sh + w0 * oa + w1 * ob + cc


def _main(xf, tif, tv, w1f, c1f, w2f, c2f, wsf, csf, nt):
    t = NPIX // nt
    grid_spec = pltpu.PrefetchScalarGridSpec(
        num_scalar_prefetch=1,
        grid=(B, t),
        in_specs=[
            pl.BlockSpec((1, CIN, nt), lambda b, j, idx: (b, 0, j)),
            pl.BlockSpec((1, 1, TOPK), lambda b, j, idx: (b, 0, 0)),
            pl.BlockSpec((1, HID, CIN), lambda b, j, idx: (idx[2 * b], 0, 0)),
            pl.BlockSpec((1, HID, CIN), lambda b, j, idx: (idx[2 * b + 1], 0, 0)),
            pl.BlockSpec((1, HID, 1), lambda b, j, idx: (idx[2 * b], 0, 0)),
            pl.BlockSpec((1, HID, 1), lambda b, j, idx: (idx[2 * b + 1], 0, 0)),
            pl.BlockSpec((1, COUT, HID), lambda b, j, idx: (idx[2 * b], 0, 0)),
            pl.BlockSpec((1, COUT, HID), lambda b, j, idx: (idx[2 * b + 1], 0, 0)),
            pl.BlockSpec((1, COUT, 1), lambda b, j, idx: (idx[2 * b], 0, 0)),
            pl.BlockSpec((1, COUT, 1), lambda b, j, idx: (idx[2 * b + 1], 0, 0)),
            pl.BlockSpec((COUT, CIN), lambda b, j, idx: (0, 0)),
            pl.BlockSpec((COUT, 1), lambda b, j, idx: (0, 0)),
        ],
        out_specs=pl.BlockSpec((1, COUT, nt), lambda b, j, idx: (b, 0, j)),
    )
    return pl.pallas_call(
        _main_kernel,
        grid_spec=grid_spec,
        out_shape=jax.ShapeDtypeStruct((B, COUT, NPIX), jnp.float32),
        compiler_params=pltpu.CompilerParams(
            dimension_semantics=("parallel", "arbitrary")),
    )(tif, xf, tv, w1f, w1f, c1f, c1f, w2f, w2f, c2f, c2f, wsf, csf)


def _fold(w, g, bb, m, v):
    s = g / jnp.sqrt(v + 1e-5)
    return w * s[..., None], (bb - m * s)


def kernel(x, r_w1, r_bn1_w, r_bn1_b, r_bn1_m, r_bn1_v, r_w2, r_bn2_w,
           r_bn2_b, r_bn2_m, r_bn2_v, s_w, s_bn_w, s_bn_b, s_bn_m, s_bn_v,
           e_w1, e_bn1_w, e_bn1_b, e_bn1_m, e_bn1_v, e_w2, e_bn2_w, e_bn2_b,
           e_bn2_m, e_bn2_v):
    xd = _pool(x)
    xdf = xd.reshape(B, CIN, NPP)

    rs1 = r_bn1_w / jnp.sqrt(r_bn1_v + 1e-5)
    w1t = (jnp.transpose(r_w1 * rs1[:, None, None, None], (2, 3, 0, 1))
           .reshape(9, RED, CIN))
    c1 = (r_bn1_b - r_bn1_m * rs1).reshape(RED, 1)
    rs2 = r_bn2_w / jnp.sqrt(r_bn2_v + 1e-5)
    w2 = r_w2[:, :, 0, 0] * rs2[:, None]
    c2 = (r_bn2_b - r_bn2_m * rs2).reshape(E, 1)
    ti, tv = _router(xdf, w1t, c1, w2, c2)
    tif = ti.reshape(B * TOPK)

    w1f, c1v = _fold(e_w1[:, :, :, 0, 0], e_bn1_w, e_bn1_b, e_bn1_m, e_bn1_v)
    c1f = c1v[:, :, None]
    w2f, c2v = _fold(e_w2[:, :, :, 0, 0], e_bn2_w, e_bn2_b, e_bn2_m, e_bn2_v)
    c2f = c2v[:, :, None]
    wsf, csv = _fold(s_w[:, :, 0, 0], s_bn_w, s_bn_b, s_bn_m, s_bn_v)
    csf = csv[:, None]
    xf = x.reshape(B, CIN, NPIX)
    out = _main(xf, tif, tv, w1f, c1f, w2f, c2f, wsf, csf, nt=3584)
    return out.reshape(B, COUT, H, W)


# speedup vs baseline: 11.7431x; 1.1435x over previous
import functools

import jax
import jax.numpy as jnp
from jax.experimental import pallas as pl
from jax.experimental.pallas import tpu as pltpu

B = 4; CIN = 96; COUT = 96; H = 224; W = 224
E = 8; TOPK = 2; HID = 192; RED = 12; POOL = 4
HP = H // POOL
NPIX = H * W
NPP = HP * HP


def _silu(v):
    return v * jax.nn.sigmoid(v)


def _pool_kernel(x_ref, o_ref):
    cb = x_ref.shape[0]
    a = x_ref[...].reshape(cb * H, W)
    r = jax.lax.broadcasted_iota(jnp.int32, (W, HP), 0)
    c = jax.lax.broadcasted_iota(jnp.int32, (W, HP), 1)
    p = jnp.where(r // POOL == c, 1.0 / (POOL * POOL), 0.0)
    aw = jnp.dot(a, p, preferred_element_type=jnp.float32)
    a3 = aw.reshape(cb, H, HP)
    rt = jax.lax.broadcasted_iota(jnp.int32, (HP, H), 0)
    ct = jax.lax.broadcasted_iota(jnp.int32, (HP, H), 1)
    pt = jnp.where(ct // POOL == rt, 1.0, 0.0)
    for ch in range(cb):
        o_ref[ch] = jnp.dot(pt, a3[ch], preferred_element_type=jnp.float32)


def _pool(x):
    x4 = x.reshape(B * CIN, H, W)
    cb = 8
    return pl.pallas_call(
        _pool_kernel,
        grid=(B * CIN // cb,),
        in_specs=[pl.BlockSpec((cb, H, W), lambda i: (i, 0, 0))],
        out_specs=pl.BlockSpec((cb, HP, HP), lambda i: (i, 0, 0)),
        out_shape=jax.ShapeDtypeStruct((B * CIN, HP, HP), jnp.float32),
        compiler_params=pltpu.CompilerParams(
            dimension_semantics=("arbitrary",)),
    )(x4)


def _router_kernel(xd_ref, w1t_ref, c1_ref, w2_ref, c2_ref, ti_ref, tv_ref):
    xd = xd_ref[0]
    pad = HP + 1
    xpad = jnp.pad(xd, ((0, 0), (pad, pad)))
    jcol = jax.lax.broadcasted_iota(jnp.int32, (1, NPP), 1) % HP
    acc = jnp.zeros((RED, NPP), jnp.float32)
    t = 0
    for di in (-1, 0, 1):
        for dj in (-1, 0, 1):
            s = HP * di + dj
            xs = xpad[:, pad + s:pad + s + NPP]
            if dj == -1:
                xs = xs * (jcol >= 1).astype(jnp.float32)
            elif dj == 1:
                xs = xs * (jcol <= HP - 2).astype(jnp.float32)
            acc = acc + jnp.dot(w1t_ref[t], xs,
                                preferred_element_type=jnp.float32)
            t += 1
    h = _silu(acc + c1_ref[...])
    m = jnp.sum(h, axis=1, keepdims=True) * (1.0 / NPP)
    gl = jnp.dot(w2_ref[...], m, preferred_element_type=jnp.float32) + c2_ref[...]
    ex = jnp.exp(gl - jnp.max(gl))
    prob = ex / jnp.sum(ex)
    ie = jax.lax.broadcasted_iota(jnp.int32, (E, 1), 0)
    v1 = jnp.max(prob)
    i1 = jnp.min(jnp.where(prob == v1, ie, E))
    p2 = jnp.where(ie == i1, -1.0, prob)
    v2 = jnp.max(p2)
    i2 = jnp.min(jnp.where(p2 == v2, ie, E))
    ssum = v1 + v2 + 1e-6
    iv = jax.lax.broadcasted_iota(jnp.int32, (1, 1, TOPK), 2)
    ti_ref[...] = jnp.where(iv == 0, i1, i2).astype(jnp.int32)
    tv_ref[...] = jnp.where(iv == 0, v1 / ssum, v2 / ssum)


def _router(xdf, w1t, c1, w2, c2):
    return pl.pallas_call(
        _router_kernel,
        grid=(B,),
        in_specs=[
            pl.BlockSpec((1, CIN, NPP), lambda b: (b, 0, 0)),
            pl.BlockSpec((9, RED, CIN), lambda b: (0, 0, 0)),
            pl.BlockSpec((RED, 1), lambda b: (0, 0)),
            pl.BlockSpec((E, RED), lambda b: (0, 0)),
            pl.BlockSpec((E, 1), lambda b: (0, 0)),
        ],
        out_specs=[
            pl.BlockSpec((1, 1, TOPK), lambda b: (b, 0, 0)),
            pl.BlockSpec((1, 1, TOPK), lambda b: (b, 0, 0)),
        ],
        out_shape=[
            jax.ShapeDtypeStruct((B, 1, TOPK), jnp.int32),
            jax.ShapeDtypeStruct((B, 1, TOPK), jnp.float32),
        ],
        compiler_params=pltpu.CompilerParams(
            dimension_semantics=("arbitrary",)),
    )(xdf, w1t, c1, w2, c2)


HT = 32


def _main_kernel(idx_ref, x_ref, tv_ref, w1a_ref, w1b_ref, c1a_ref, c1b_ref,
                 w2a_ref, w2b_ref, c2a_ref, c2b_ref, ws_ref, cs_ref, o_ref):
    w0 = tv_ref[0, 0, 0]
    w1 = tv_ref[0, 0, 1]
    wup = jnp.concatenate([w1a_ref[0], w1b_ref[0], ws_ref[...]], axis=0)
    bup = jnp.concatenate([c1a_ref[0], c1b_ref[0], cs_ref[...]], axis=0)
    wdn = jnp.concatenate([w0 * w2a_ref[0], w1 * w2b_ref[0]], axis=1)
    cc = w0 * c2a_ref[0] + w1 * c2b_ref[0]
    for r in range(HT):
        xr = x_ref[0, :, r, :]
        u = jnp.dot(wup, xr, preferred_element_type=jnp.float32) + bup
        v = _silu(u)
        o_ref[0, :, r, :] = (v[2 * HID:] + cc +
                             jnp.dot(wdn, v[:2 * HID],
                                     preferred_element_type=jnp.float32))


def _main(x, tif, tv, w1f, c1f, w2f, c2f, wsf, csf):
    grid_spec = pltpu.PrefetchScalarGridSpec(
        num_scalar_prefetch=1,
        grid=(B, H // HT),
        in_specs=[
            pl.BlockSpec((1, CIN, HT, W), lambda b, j, idx: (b, 0, j, 0)),
            pl.BlockSpec((1, 1, TOPK), lambda b, j, idx: (b, 0, 0)),
            pl.BlockSpec((1, HID, CIN), lambda b, j, idx: (idx[2 * b], 0, 0)),
            pl.BlockSpec((1, HID, CIN), lambda b, j, idx: (idx[2 * b + 1], 0, 0)),
            pl.BlockSpec((1, HID, 1), lambda b, j, idx: (idx[2 * b], 0, 0)),
            pl.BlockSpec((1, HID, 1), lambda b, j, idx: (idx[2 * b + 1], 0, 0)),
            pl.BlockSpec((1, COUT, HID), lambda b, j, idx: (idx[2 * b], 0, 0)),
            pl.BlockSpec((1, COUT, HID), lambda b, j, idx: (idx[2 * b + 1], 0, 0)),
            pl.BlockSpec((1, COUT, 1), lambda b, j, idx: (idx[2 * b], 0, 0)),
            pl.BlockSpec((1, COUT, 1), lambda b, j, idx: (idx[2 * b + 1], 0, 0)),
            pl.BlockSpec((COUT, CIN), lambda b, j, idx: (0, 0)),
            pl.BlockSpec((COUT, 1), lambda b, j, idx: (0, 0)),
        ],
        out_specs=pl.BlockSpec((1, COUT, HT, W), lambda b, j, idx: (b, 0, j, 0)),
    )
    return pl.pallas_call(
        _main_kernel,
        grid_spec=grid_spec,
        out_shape=jax.ShapeDtypeStruct((B, COUT, H, W), jnp.float32),
        compiler_params=pltpu.CompilerParams(
            dimension_semantics=("parallel", "arbitrary")),
    )(tif, x, tv, w1f, w1f, c1f, c1f, w2f, w2f, c2f, c2f, wsf, csf)


def _fold(w, g, bb, m, v):
    s = g / jnp.sqrt(v + 1e-5)
    return w * s[..., None], (bb - m * s)


def kernel(x, r_w1, r_bn1_w, r_bn1_b, r_bn1_m, r_bn1_v, r_w2, r_bn2_w,
           r_bn2_b, r_bn2_m, r_bn2_v, s_w, s_bn_w, s_bn_b, s_bn_m, s_bn_v,
           e_w1, e_bn1_w, e_bn1_b, e_bn1_m, e_bn1_v, e_w2, e_bn2_w, e_bn2_b,
           e_bn2_m, e_bn2_v):
    xd = _pool(x)
    xdf = xd.reshape(B, CIN, NPP)

    rs1 = r_bn1_w / jnp.sqrt(r_bn1_v + 1e-5)
    w1t = (jnp.transpose(r_w1 * rs1[:, None, None, None], (2, 3, 0, 1))
           .reshape(9, RED, CIN))
    c1 = (r_bn1_b - r_bn1_m * rs1).reshape(RED, 1)
    rs2 = r_bn2_w / jnp.sqrt(r_bn2_v + 1e-5)
    w2 = r_w2[:, :, 0, 0] * rs2[:, None]
    c2 = (r_bn2_b - r_bn2_m * rs2).reshape(E, 1)
    ti, tv = _router(xdf, w1t, c1, w2, c2)
    tif = ti.reshape(B * TOPK)

    w1f, c1v = _fold(e_w1[:, :, :, 0, 0], e_bn1_w, e_bn1_b, e_bn1_m, e_bn1_v)
    c1f = c1v[:, :, None]
    w2f, c2v = _fold(e_w2[:, :, :, 0, 0], e_bn2_w, e_bn2_b, e_bn2_m, e_bn2_v)
    c2f = c2v[:, :, None]
    wsf, csv = _fold(s_w[:, :, 0, 0], s_bn_w, s_bn_b, s_bn_m, s_bn_v)
    csf = csv[:, None]
    return _main(x, tif, tv, w1f, c1f, w2f, c2f, wsf, csf)


# speedup vs baseline: 12.9908x; 1.1063x over previous
import functools

import jax
import jax.numpy as jnp
from jax.experimental import pallas as pl
from jax.experimental.pallas import tpu as pltpu

B = 4; CIN = 96; COUT = 96; H = 224; W = 224
E = 8; TOPK = 2; HID = 192; RED = 12; POOL = 4
HP = H // POOL
NPIX = H * W
NPP = HP * HP


def _silu(v):
    return v * jax.nn.sigmoid(v)


def _pool_kernel(x_ref, o_ref, o2_ref):
    cb = x_ref.shape[0]
    xb = x_ref[...]
    o2_ref[...] = xb.astype(jnp.bfloat16).reshape(cb, H * W)
    a = xb.reshape(cb * H, W)
    r = jax.lax.broadcasted_iota(jnp.int32, (W, HP), 0)
    c = jax.lax.broadcasted_iota(jnp.int32, (W, HP), 1)
    p = jnp.where(r // POOL == c, 1.0 / (POOL * POOL), 0.0)
    aw = jnp.dot(a, p, preferred_element_type=jnp.float32)
    a3 = aw.reshape(cb, H, HP)
    rt = jax.lax.broadcasted_iota(jnp.int32, (HP, H), 0)
    ct = jax.lax.broadcasted_iota(jnp.int32, (HP, H), 1)
    pt = jnp.where(ct // POOL == rt, 1.0, 0.0)
    for ch in range(cb):
        o_ref[ch] = jnp.dot(pt, a3[ch], preferred_element_type=jnp.float32)


def _pool(x):
    x4 = x.reshape(B * CIN, H, W)
    cb = 32
    return pl.pallas_call(
        _pool_kernel,
        grid=(B * CIN // cb,),
        in_specs=[pl.BlockSpec((cb, H, W), lambda i: (i, 0, 0))],
        out_specs=[
            pl.BlockSpec((cb, HP, HP), lambda i: (i, 0, 0)),
            pl.BlockSpec((cb, NPIX), lambda i: (i, 0)),
        ],
        out_shape=[
            jax.ShapeDtypeStruct((B * CIN, HP, HP), jnp.float32),
            jax.ShapeDtypeStruct((B * CIN, NPIX), jnp.bfloat16),
        ],
        compiler_params=pltpu.CompilerParams(
            dimension_semantics=("arbitrary",)),
    )(x4)


def _router_kernel(xd_ref, w1t_ref, c1_ref, w2_ref, c2_ref, ti_ref, tv_ref):
    xd = xd_ref[0]
    pad = HP + 1
    xpad = jnp.pad(xd, ((0, 0), (pad, pad)))
    jcol = jax.lax.broadcasted_iota(jnp.int32, (1, NPP), 1) % HP
    acc = jnp.zeros((RED, NPP), jnp.float32)
    t = 0
    for di in (-1, 0, 1):
        for dj in (-1, 0, 1):
            s = HP * di + dj
            xs = xpad[:, pad + s:pad + s + NPP]
            if dj == -1:
                xs = xs * (jcol >= 1).astype(jnp.float32)
            elif dj == 1:
                xs = xs * (jcol <= HP - 2).astype(jnp.float32)
            acc = acc + jnp.dot(w1t_ref[t], xs,
                                preferred_element_type=jnp.float32)
            t += 1
    h = _silu(acc + c1_ref[...])
    m = jnp.sum(h, axis=1, keepdims=True) * (1.0 / NPP)
    gl = jnp.dot(w2_ref[...], m, preferred_element_type=jnp.float32) + c2_ref[...]
    ex = jnp.exp(gl - jnp.max(gl))
    prob = ex / jnp.sum(ex)
    ie = jax.lax.broadcasted_iota(jnp.int32, (E, 1), 0)
    v1 = jnp.max(prob)
    i1 = jnp.min(jnp.where(prob == v1, ie, E))
    p2 = jnp.where(ie == i1, -1.0, prob)
    v2 = jnp.max(p2)
    i2 = jnp.min(jnp.where(p2 == v2, ie, E))
    ssum = v1 + v2 + 1e-6
    iv = jax.lax.broadcasted_iota(jnp.int32, (1, 1, TOPK), 2)
    ti_ref[...] = jnp.where(iv == 0, i1, i2).astype(jnp.int32)
    tv_ref[...] = jnp.where(iv == 0, v1 / ssum, v2 / ssum)


def _router(xdf, w1t, c1, w2, c2):
    return pl.pallas_call(
        _router_kernel,
        grid=(B,),
        in_specs=[
            pl.BlockSpec((1, CIN, NPP), lambda b: (b, 0, 0)),
            pl.BlockSpec((9, RED, CIN), lambda b: (0, 0, 0)),
            pl.BlockSpec((RED, 1), lambda b: (0, 0)),
            pl.BlockSpec((E, RED), lambda b: (0, 0)),
            pl.BlockSpec((E, 1), lambda b: (0, 0)),
        ],
        out_specs=[
            pl.BlockSpec((1, 1, TOPK), lambda b: (b, 0, 0)),
            pl.BlockSpec((1, 1, TOPK), lambda b: (b, 0, 0)),
        ],
        out_shape=[
            jax.ShapeDtypeStruct((B, 1, TOPK), jnp.int32),
            jax.ShapeDtypeStruct((B, 1, TOPK), jnp.float32),
        ],
        compiler_params=pltpu.CompilerParams(
            dimension_semantics=("arbitrary",)),
    )(xdf, w1t, c1, w2, c2)


HT = 16
NT = HT * W


def _main_kernel(idx_ref, x_ref, tv_ref, w1a_ref, w1b_ref, c1a_ref, c1b_ref,
                 w2a_ref, w2b_ref, c2a_ref, c2b_ref, ws_ref, cs_ref, o_ref):
    w0 = tv_ref[0, 0, 0]
    w1 = tv_ref[0, 0, 1]
    wup = jnp.concatenate([w1a_ref[0], w1b_ref[0], ws_ref[...]], axis=0)
    bup = jnp.concatenate([c1a_ref[0], c1b_ref[0], cs_ref[...]], axis=0)
    wdn = jnp.concatenate([w0 * w2a_ref[0], w1 * w2b_ref[0]], axis=1)
    cc = w0 * c2a_ref[0] + w1 * c2b_ref[0]
    wup16 = wup.astype(jnp.bfloat16)
    wdn16 = wdn.astype(jnp.bfloat16)
    xt = x_ref[0]
    u = jnp.dot(wup16, xt, preferred_element_type=jnp.float32) + bup
    v = _silu(u)
    res = (v[2 * HID:] + cc +
           jnp.dot(wdn16, v[:2 * HID].astype(jnp.bfloat16),
                   preferred_element_type=jnp.float32))
    for r in range(HT):
        o_ref[0, :, r, :] = res[:, r * W:(r + 1) * W]


def _main(xf16, tif, tv, w1f, c1f, w2f, c2f, wsf, csf):
    grid_spec = pltpu.PrefetchScalarGridSpec(
        num_scalar_prefetch=1,
        grid=(B, H // HT),
        in_specs=[
            pl.BlockSpec((1, CIN, NT), lambda b, j, idx: (b, 0, j)),
            pl.BlockSpec((1, 1, TOPK), lambda b, j, idx: (b, 0, 0)),
            pl.BlockSpec((1, HID, CIN), lambda b, j, idx: (idx[2 * b], 0, 0)),
            pl.BlockSpec((1, HID, CIN), lambda b, j, idx: (idx[2 * b + 1], 0, 0)),
            pl.BlockSpec((1, HID, 1), lambda b, j, idx: (idx[2 * b], 0, 0)),
            pl.BlockSpec((1, HID, 1), lambda b, j, idx: (idx[2 * b + 1], 0, 0)),
            pl.BlockSpec((1, COUT, HID), lambda b, j, idx: (idx[2 * b], 0, 0)),
            pl.BlockSpec((1, COUT, HID), lambda b, j, idx: (idx[2 * b + 1], 0, 0)),
            pl.BlockSpec((1, COUT, 1), lambda b, j, idx: (idx[2 * b], 0, 0)),
            pl.BlockSpec((1, COUT, 1), lambda b, j, idx: (idx[2 * b + 1], 0, 0)),
            pl.BlockSpec((COUT, CIN), lambda b, j, idx: (0, 0)),
            pl.BlockSpec((COUT, 1), lambda b, j, idx: (0, 0)),
        ],
        out_specs=pl.BlockSpec((1, COUT, HT, W), lambda b, j, idx: (b, 0, j, 0)),
    )
    return pl.pallas_call(
        _main_kernel,
        grid_spec=grid_spec,
        out_shape=jax.ShapeDtypeStruct((B, COUT, H, W), jnp.float32),
        compiler_params=pltpu.CompilerParams(
            dimension_semantics=("parallel", "arbitrary")),
    )(tif, xf16, tv, w1f, w1f, c1f, c1f, w2f, w2f, c2f, c2f, wsf, csf)


def _fold(w, g, bb, m, v):
    s = g / jnp.sqrt(v + 1e-5)
    return w * s[..., None], (bb - m * s)


def kernel(x, r_w1, r_bn1_w, r_bn1_b, r_bn1_m, r_bn1_v, r_w2, r_bn2_w,
           r_bn2_b, r_bn2_m, r_bn2_v, s_w, s_bn_w, s_bn_b, s_bn_m, s_bn_v,
           e_w1, e_bn1_w, e_bn1_b, e_bn1_m, e_bn1_v, e_w2, e_bn2_w, e_bn2_b,
           e_bn2_m, e_bn2_v):
    xd, xflat16 = _pool(x)
    xdf = xd.reshape(B, CIN, NPP)

    rs1 = r_bn1_w / jnp.sqrt(r_bn1_v + 1e-5)
    w1t = (jnp.transpose(r_w1 * rs1[:, None, None, None], (2, 3, 0, 1))
           .reshape(9, RED, CIN))
    c1 = (r_bn1_b - r_bn1_m * rs1).reshape(RED, 1)
    rs2 = r_bn2_w / jnp.sqrt(r_bn2_v + 1e-5)
    w2 = r_w2[:, :, 0, 0] * rs2[:, None]
    c2 = (r_bn2_b - r_bn2_m * rs2).reshape(E, 1)
    ti, tv = _router(xdf, w1t, c1, w2, c2)
    tif = ti.reshape(B * TOPK)

    w1f, c1v = _fold(e_w1[:, :, :, 0, 0], e_bn1_w, e_bn1_b, e_bn1_m, e_bn1_v)
    c1f = c1v[:, :, None]
    w2f, c2v = _fold(e_w2[:, :, :, 0, 0], e_bn2_w, e_bn2_b, e_bn2_m, e_bn2_v)
    c2f = c2v[:, :, None]
    wsf, csv = _fold(s_w[:, :, 0, 0], s_bn_w, s_bn_b, s_bn_m, s_bn_v)
    csf = csv[:, None]
    return _main(xflat16.reshape(B, CIN, NPIX), tif, tv, w1f, c1f, w2f, c2f,
                 wsf, csf)


# speedup vs baseline: 14.7677x; 1.1368x over previous
import functools

import jax
import jax.numpy as jnp
from jax.experimental import pallas as pl
from jax.experimental.pallas import tpu as pltpu

B = 4; CIN = 96; COUT = 96; H = 224; W = 224
E = 8; TOPK = 2; HID = 192; RED = 12; POOL = 4
HP = H // POOL
NPIX = H * W
NPP = HP * HP


def _silu(v):
    return v * jax.nn.sigmoid(v)


def _pool_kernel(x_ref, o_ref, o2_ref):
    cb = x_ref.shape[0]
    xb = x_ref[...]
    o2_ref[...] = xb.astype(jnp.bfloat16).reshape(cb, H * W)
    a = xb.reshape(cb * H, W)
    r = jax.lax.broadcasted_iota(jnp.int32, (W, HP), 0)
    c = jax.lax.broadcasted_iota(jnp.int32, (W, HP), 1)
    p = jnp.where(r // POOL == c, 1.0 / (POOL * POOL), 0.0)
    aw = jnp.dot(a, p, preferred_element_type=jnp.float32)
    a3 = aw.reshape(cb, H, HP)
    rt = jax.lax.broadcasted_iota(jnp.int32, (HP, H), 0)
    ct = jax.lax.broadcasted_iota(jnp.int32, (HP, H), 1)
    pt = jnp.where(ct // POOL == rt, 1.0, 0.0)
    for ch in range(cb):
        o_ref[ch] = jnp.dot(pt, a3[ch], preferred_element_type=jnp.float32)


def _pool(x):
    x4 = x.reshape(B * CIN, H, W)
    cb = 32
    return pl.pallas_call(
        _pool_kernel,
        grid=(B * CIN // cb,),
        in_specs=[pl.BlockSpec((cb, H, W), lambda i: (i, 0, 0))],
        out_specs=[
            pl.BlockSpec((cb, HP, HP), lambda i: (i, 0, 0)),
            pl.BlockSpec((cb, NPIX), lambda i: (i, 0)),
        ],
        out_shape=[
            jax.ShapeDtypeStruct((B * CIN, HP, HP), jnp.float32),
            jax.ShapeDtypeStruct((B * CIN, NPIX), jnp.bfloat16),
        ],
        compiler_params=pltpu.CompilerParams(
            dimension_semantics=("arbitrary",)),
    )(x4)


def _router_kernel(xd_ref, w1t_ref, c1_ref, w2_ref, c2_ref, ti_ref, tv_ref):
    xd = xd_ref[0]
    pad = HP + 1
    xpad = jnp.pad(xd, ((0, 0), (pad, pad)))
    jcol = jax.lax.broadcasted_iota(jnp.int32, (1, NPP), 1) % HP
    acc = jnp.zeros((RED, NPP), jnp.float32)
    t = 0
    for di in (-1, 0, 1):
        for dj in (-1, 0, 1):
            s = HP * di + dj
            xs = xpad[:, pad + s:pad + s + NPP]
            if dj == -1:
                xs = xs * (jcol >= 1).astype(jnp.float32)
            elif dj == 1:
                xs = xs * (jcol <= HP - 2).astype(jnp.float32)
            acc = acc + jnp.dot(w1t_ref[t], xs,
                                preferred_element_type=jnp.float32)
            t += 1
    h = _silu(acc + c1_ref[...])
    m = jnp.sum(h, axis=1, keepdims=True) * (1.0 / NPP)
    gl = jnp.dot(w2_ref[...], m, preferred_element_type=jnp.float32) + c2_ref[...]
    ex = jnp.exp(gl - jnp.max(gl))
    prob = ex / jnp.sum(ex)
    ie = jax.lax.broadcasted_iota(jnp.int32, (E, 1), 0)
    v1 = jnp.max(prob)
    i1 = jnp.min(jnp.where(prob == v1, ie, E))
    p2 = jnp.where(ie == i1, -1.0, prob)
    v2 = jnp.max(p2)
    i2 = jnp.min(jnp.where(p2 == v2, ie, E))
    ssum = v1 + v2 + 1e-6
    iv = jax.lax.broadcasted_iota(jnp.int32, (1, 1, TOPK), 2)
    ti_ref[...] = jnp.where(iv == 0, i1, i2).astype(jnp.int32)
    tv_ref[...] = jnp.where(iv == 0, v1 / ssum, v2 / ssum)


def _router(xdf, w1t, c1, w2, c2):
    return pl.pallas_call(
        _router_kernel,
        grid=(B,),
        in_specs=[
            pl.BlockSpec((1, CIN, NPP), lambda b: (b, 0, 0)),
            pl.BlockSpec((9, RED, CIN), lambda b: (0, 0, 0)),
            pl.BlockSpec((RED, 1), lambda b: (0, 0)),
            pl.BlockSpec((E, RED), lambda b: (0, 0)),
            pl.BlockSpec((E, 1), lambda b: (0, 0)),
        ],
        out_specs=[
            pl.BlockSpec((1, 1, TOPK), lambda b: (b, 0, 0)),
            pl.BlockSpec((1, 1, TOPK), lambda b: (b, 0, 0)),
        ],
        out_shape=[
            jax.ShapeDtypeStruct((B, 1, TOPK), jnp.int32),
            jax.ShapeDtypeStruct((B, 1, TOPK), jnp.float32),
        ],
        compiler_params=pltpu.CompilerParams(
            dimension_semantics=("arbitrary",)),
    )(xdf, w1t, c1, w2, c2)


HT = 32
NT = HT * W


def _main_kernel(idx_ref, x_ref, tv_ref, w1a_ref, w1b_ref, c1a_ref, c1b_ref,
                 w2a_ref, w2b_ref, c2a_ref, c2b_ref, ws_ref, cs_ref, o_ref):
    w0 = tv_ref[0, 0, 0]
    w1 = tv_ref[0, 0, 1]
    wup = jnp.concatenate([w1a_ref[0], w1b_ref[0], ws_ref[...]], axis=0)
    bup = jnp.concatenate([c1a_ref[0], c1b_ref[0], cs_ref[...]], axis=0)
    wdn = jnp.concatenate([w0 * w2a_ref[0], w1 * w2b_ref[0]], axis=1)
    cc = w0 * c2a_ref[0] + w1 * c2b_ref[0]
    wup16 = wup.astype(jnp.bfloat16)
    wdn16 = wdn.astype(jnp.bfloat16)
    xt = x_ref[0]
    u = (jnp.dot(wup16, xt, preferred_element_type=jnp.float32)
         + bup).astype(jnp.bfloat16)
    v = _silu(u)
    res = (v[2 * HID:].astype(jnp.float32) + cc +
           jnp.dot(wdn16, v[:2 * HID],
                   preferred_element_type=jnp.float32))

    for r in range(HT):
        o_ref[0, :, r, :] = res[:, r * W:(r + 1) * W]


def _main(xf16, tif, tv, w1f, c1f, w2f, c2f, wsf, csf):
    grid_spec = pltpu.PrefetchScalarGridSpec(
        num_scalar_prefetch=1,
        grid=(B, H // HT),
        in_specs=[
            pl.BlockSpec((1, CIN, NT), lambda b, j, idx: (b, 0, j)),
            pl.BlockSpec((1, 1, TOPK), lambda b, j, idx: (b, 0, 0)),
            pl.BlockSpec((1, HID, CIN), lambda b, j, idx: (idx[2 * b], 0, 0)),
            pl.BlockSpec((1, HID, CIN), lambda b, j, idx: (idx[2 * b + 1], 0, 0)),
            pl.BlockSpec((1, HID, 1), lambda b, j, idx: (idx[2 * b], 0, 0)),
            pl.BlockSpec((1, HID, 1), lambda b, j, idx: (idx[2 * b + 1], 0, 0)),
            pl.BlockSpec((1, COUT, HID), lambda b, j, idx: (idx[2 * b], 0, 0)),
            pl.BlockSpec((1, COUT, HID), lambda b, j, idx: (idx[2 * b + 1], 0, 0)),
            pl.BlockSpec((1, COUT, 1), lambda b, j, idx: (idx[2 * b], 0, 0)),
            pl.BlockSpec((1, COUT, 1), lambda b, j, idx: (idx[2 * b + 1], 0, 0)),
            pl.BlockSpec((COUT, CIN), lambda b, j, idx: (0, 0)),
            pl.BlockSpec((COUT, 1), lambda b, j, idx: (0, 0)),
        ],
        out_specs=pl.BlockSpec((1, COUT, HT, W), lambda b, j, idx: (b, 0, j, 0)),
    )
    return pl.pallas_call(
        _main_kernel,
        grid_spec=grid_spec,
        out_shape=jax.ShapeDtypeStruct((B, COUT, H, W), jnp.float32),
        compiler_params=pltpu.CompilerParams(
            dimension_semantics=("parallel", "arbitrary")),
    )(tif, xf16, tv, w1f, w1f, c1f, c1f, w2f, w2f, c2f, c2f, wsf, csf)


def _fold(w, g, bb, m, v):
    s = g / jnp.sqrt(v + 1e-5)
    return w * s[..., None], (bb - m * s)


def kernel(x, r_w1, r_bn1_w, r_bn1_b, r_bn1_m, r_bn1_v, r_w2, r_bn2_w,
           r_bn2_b, r_bn2_m, r_bn2_v, s_w, s_bn_w, s_bn_b, s_bn_m, s_bn_v,
           e_w1, e_bn1_w, e_bn1_b, e_bn1_m, e_bn1_v, e_w2, e_bn2_w, e_bn2_b,
           e_bn2_m, e_bn2_v):
    xd, xflat16 = _pool(x)
    xdf = xd.reshape(B, CIN, NPP)

    rs1 = r_bn1_w / jnp.sqrt(r_bn1_v + 1e-5)
    w1t = (jnp.transpose(r_w1 * rs1[:, None, None, None], (2, 3, 0, 1))
           .reshape(9, RED, CIN))
    c1 = (r_bn1_b - r_bn1_m * rs1).reshape(RED, 1)
    rs2 = r_bn2_w / jnp.sqrt(r_bn2_v + 1e-5)
    w2 = r_w2[:, :, 0, 0] * rs2[:, None]
    c2 = (r_bn2_b - r_bn2_m * rs2).reshape(E, 1)
    ti, tv = _router(xdf, w1t, c1, w2, c2)
    tif = ti.reshape(B * TOPK)

    w1f, c1v = _fold(e_w1[:, :, :, 0, 0], e_bn1_w, e_bn1_b, e_bn1_m, e_bn1_v)
    c1f = c1v[:, :, None]
    w2f, c2v = _fold(e_w2[:, :, :, 0, 0], e_bn2_w, e_bn2_b, e_bn2_m, e_bn2_v)
    c2f = c2v[:, :, None]
    wsf, csv = _fold(s_w[:, :, 0, 0], s_bn_w, s_bn_b, s_bn_m, s_bn_v)
    csf = csv[:, None]
    return _main(xflat16.reshape(B, CIN, NPIX), tif, tv, w1f, c1f, w2f, c2f,
                 wsf, csf)


# speedup vs baseline: 15.5321x; 1.0518x over previous
import functools

import jax
import jax.numpy as jnp
from jax.experimental import pallas as pl
from jax.experimental.pallas import tpu as pltpu

B = 4; CIN = 96; COUT = 96; H = 224; W = 224
E = 8; TOPK = 2; HID = 192; RED = 12; POOL = 4
HP = H // POOL
NPIX = H * W
NPP = HP * HP


def _silu(v):
    return v * jax.nn.sigmoid(v)


def _pool_kernel(x_ref, o_ref, o2_ref):
    cb = x_ref.shape[0]
    xb = x_ref[...]
    o2_ref[...] = xb.astype(jnp.bfloat16).reshape(cb, H * W)
    a = xb.reshape(cb * H, W)
    r = jax.lax.broadcasted_iota(jnp.int32, (W, HP), 0)
    c = jax.lax.broadcasted_iota(jnp.int32, (W, HP), 1)
    p = jnp.where(r // POOL == c, 1.0 / (POOL * POOL), 0.0)
    aw = jnp.dot(a, p, preferred_element_type=jnp.float32)
    a3 = aw.reshape(cb, H, HP)
    rt = jax.lax.broadcasted_iota(jnp.int32, (HP, H), 0)
    ct = jax.lax.broadcasted_iota(jnp.int32, (HP, H), 1)
    pt = jnp.where(ct // POOL == rt, 1.0, 0.0)
    pooled = jnp.stack(
        [jnp.dot(pt, a3[ch], preferred_element_type=jnp.float32)
         for ch in range(cb)], axis=0)
    o_ref[...] = pooled.reshape(cb, NPP)


def _pool(x):
    x4 = x.reshape(B * CIN, H, W)
    cb = 32
    return pl.pallas_call(
        _pool_kernel,
        grid=(B * CIN // cb,),
        in_specs=[pl.BlockSpec((cb, H, W), lambda i: (i, 0, 0))],
        out_specs=[
            pl.BlockSpec((cb, NPP), lambda i: (i, 0)),
            pl.BlockSpec((cb, NPIX), lambda i: (i, 0)),
        ],
        out_shape=[
            jax.ShapeDtypeStruct((B * CIN, NPP), jnp.float32),
            jax.ShapeDtypeStruct((B * CIN, NPIX), jnp.bfloat16),
        ],
        compiler_params=pltpu.CompilerParams(
            dimension_semantics=("arbitrary",)),
    )(x4)


def _router_kernel(xd_ref, w1t_ref, c1_ref, w2_ref, c2_ref, ti_ref, tv_ref):
    xd = xd_ref[0]
    pad = HP + 1
    jcol = jax.lax.broadcasted_iota(jnp.int32, (1, NPP), 1) % HP
    acc = jnp.zeros((RED, NPP), jnp.float32)
    t = 0
    for di in (-1, 0, 1):
        for dj in (-1, 0, 1):
            s = HP * di + dj
            y = jnp.dot(w1t_ref[t], xd, preferred_element_type=jnp.float32)
            ys = jnp.pad(y, ((0, 0), (pad, pad)))[:, pad + s:pad + s + NPP]
            if dj == -1:
                ys = ys * (jcol >= 1).astype(jnp.float32)
            elif dj == 1:
                ys = ys * (jcol <= HP - 2).astype(jnp.float32)
            acc = acc + ys
            t += 1
    h = _silu(acc + c1_ref[...])
    m = jnp.sum(h, axis=1, keepdims=True) * (1.0 / NPP)
    gl = jnp.dot(w2_ref[...], m, preferred_element_type=jnp.float32) + c2_ref[...]
    ex = jnp.exp(gl - jnp.max(gl))
    prob = ex / jnp.sum(ex)
    ie = jax.lax.broadcasted_iota(jnp.int32, (E, 1), 0)
    v1 = jnp.max(prob)
    i1 = jnp.min(jnp.where(prob == v1, ie, E))
    p2 = jnp.where(ie == i1, -1.0, prob)
    v2 = jnp.max(p2)
    i2 = jnp.min(jnp.where(p2 == v2, ie, E))
    ssum = v1 + v2 + 1e-6
    iv = jax.lax.broadcasted_iota(jnp.int32, (1, 1, TOPK), 2)
    ti_ref[...] = jnp.where(iv == 0, i1, i2).astype(jnp.int32)
    tv_ref[...] = jnp.where(iv == 0, v1 / ssum, v2 / ssum)


def _router(xdf, w1t, c1, w2, c2):
    return pl.pallas_call(
        _router_kernel,
        grid=(B,),
        in_specs=[
            pl.BlockSpec((1, CIN, NPP), lambda b: (b, 0, 0)),
            pl.BlockSpec((9, RED, CIN), lambda b: (0, 0, 0)),
            pl.BlockSpec((RED, 1), lambda b: (0, 0)),
            pl.BlockSpec((E, RED), lambda b: (0, 0)),
            pl.BlockSpec((E, 1), lambda b: (0, 0)),
        ],
        out_specs=[
            pl.BlockSpec((1, 1, TOPK), lambda b: (b, 0, 0)),
            pl.BlockSpec((1, 1, TOPK), lambda b: (b, 0, 0)),
        ],
        out_shape=[
            jax.ShapeDtypeStruct((B, 1, TOPK), jnp.int32),
            jax.ShapeDtypeStruct((B, 1, TOPK), jnp.float32),
        ],
        compiler_params=pltpu.CompilerParams(
            dimension_semantics=("arbitrary",)),
    )(xdf, w1t, c1, w2, c2)


HT = 32
NT = HT * W


def _main_kernel(idx_ref, x_ref, tv_ref, w1a_ref, w1b_ref, c1a_ref, c1b_ref,
                 w2a_ref, w2b_ref, c2a_ref, c2b_ref, ws_ref, cs_ref, o_ref):
    w0 = tv_ref[0, 0, 0]
    w1 = tv_ref[0, 0, 1]
    wup = jnp.concatenate([w1a_ref[0], w1b_ref[0], ws_ref[...]], axis=0)
    bup = jnp.concatenate([c1a_ref[0], c1b_ref[0], cs_ref[...]], axis=0)
    wdn = jnp.concatenate([w0 * w2a_ref[0], w1 * w2b_ref[0]], axis=1)
    cc = w0 * c2a_ref[0] + w1 * c2b_ref[0]
    wup16 = wup.astype(jnp.bfloat16)
    wdn16 = wdn.astype(jnp.bfloat16)
    xt = x_ref[0]
    u = (jnp.dot(wup16, xt, preferred_element_type=jnp.float32)
         + bup).astype(jnp.bfloat16)
    v = _silu(u)
    res = (v[2 * HID:].astype(jnp.float32) + cc +
           jnp.dot(wdn16, v[:2 * HID],
                   preferred_element_type=jnp.float32))

    for r in range(HT):
        o_ref[0, :, r, :] = res[:, r * W:(r + 1) * W]


def _main(xf16, tif, tv, w1f, c1f, w2f, c2f, wsf, csf):
    grid_spec = pltpu.PrefetchScalarGridSpec(
        num_scalar_prefetch=1,
        grid=(B, H // HT),
        in_specs=[
            pl.BlockSpec((1, CIN, NT), lambda b, j, idx: (b, 0, j)),
            pl.BlockSpec((1, 1, TOPK), lambda b, j, idx: (b, 0, 0)),
            pl.BlockSpec((1, HID, CIN), lambda b, j, idx: (idx[2 * b], 0, 0)),
            pl.BlockSpec((1, HID, CIN), lambda b, j, idx: (idx[2 * b + 1], 0, 0)),
            pl.BlockSpec((1, HID, 1), lambda b, j, idx: (idx[2 * b], 0, 0)),
            pl.BlockSpec((1, HID, 1), lambda b, j, idx: (idx[2 * b + 1], 0, 0)),
            pl.BlockSpec((1, COUT, HID), lambda b, j, idx: (idx[2 * b], 0, 0)),
            pl.BlockSpec((1, COUT, HID), lambda b, j, idx: (idx[2 * b + 1], 0, 0)),
            pl.BlockSpec((1, COUT, 1), lambda b, j, idx: (idx[2 * b], 0, 0)),
            pl.BlockSpec((1, COUT, 1), lambda b, j, idx: (idx[2 * b + 1], 0, 0)),
            pl.BlockSpec((COUT, CIN), lambda b, j, idx: (0, 0)),
            pl.BlockSpec((COUT, 1), lambda b, j, idx: (0, 0)),
        ],
        out_specs=pl.BlockSpec((1, COUT, HT, W), lambda b, j, idx: (b, 0, j, 0)),
    )
    return pl.pallas_call(
        _main_kernel,
        grid_spec=grid_spec,
        out_shape=jax.ShapeDtypeStruct((B, COUT, H, W), jnp.float32),
        compiler_params=pltpu.CompilerParams(
            dimension_semantics=("parallel", "arbitrary")),
    )(tif, xf16, tv, w1f, w1f, c1f, c1f, w2f, w2f, c2f, c2f, wsf, csf)


def _fold(w, g, bb, m, v):
    s = g / jnp.sqrt(v + 1e-5)
    return w * s[..., None], (bb - m * s)


def kernel(x, r_w1, r_bn1_w, r_bn1_b, r_bn1_m, r_bn1_v, r_w2, r_bn2_w,
           r_bn2_b, r_bn2_m, r_bn2_v, s_w, s_bn_w, s_bn_b, s_bn_m, s_bn_v,
           e_w1, e_bn1_w, e_bn1_b, e_bn1_m, e_bn1_v, e_w2, e_bn2_w, e_bn2_b,
           e_bn2_m, e_bn2_v):
    xd, xflat16 = _pool(x)
    xdf = xd.reshape(B, CIN, NPP)

    rs1 = r_bn1_w / jnp.sqrt(r_bn1_v + 1e-5)
    w1t = (jnp.transpose(r_w1 * rs1[:, None, None, None], (2, 3, 0, 1))
           .reshape(9, RED, CIN))
    c1 = (r_bn1_b - r_bn1_m * rs1).reshape(RED, 1)
    rs2 = r_bn2_w / jnp.sqrt(r_bn2_v + 1e-5)
    w2 = r_w2[:, :, 0, 0] * rs2[:, None]
    c2 = (r_bn2_b - r_bn2_m * rs2).reshape(E, 1)
    ti, tv = _router(xdf, w1t, c1, w2, c2)
    tif = ti.reshape(B * TOPK)

    w1f, c1v = _fold(e_w1[:, :, :, 0, 0], e_bn1_w, e_bn1_b, e_bn1_m, e_bn1_v)
    c1f = c1v[:, :, None]
    w2f, c2v = _fold(e_w2[:, :, :, 0, 0], e_bn2_w, e_bn2_b, e_bn2_m, e_bn2_v)
    c2f = c2v[:, :, None]
    wsf, csv = _fold(s_w[:, :, 0, 0], s_bn_w, s_bn_b, s_bn_m, s_bn_v)
    csf = csv[:, None]
    return _main(xflat16.reshape(B, CIN, NPIX), tif, tv, w1f, c1f, w2f, c2f,
                 wsf, csf)


# speedup vs baseline: 16.1575x; 1.0403x over previous
import functools

import jax
import jax.numpy as jnp
from jax.experimental import pallas as pl
from jax.experimental.pallas import tpu as pltpu

B = 4; CIN = 96; COUT = 96; H = 224; W = 224
E = 8; TOPK = 2; HID = 192; RED = 12; POOL = 4
HP = H // POOL
NPIX = H * W
NPP = HP * HP


def _silu(v):
    return v * jax.nn.sigmoid(v)


def _pool_kernel(x_ref, o_ref, o2_ref):
    cb = x_ref.shape[0]
    xb = x_ref[...]
    o2_ref[...] = xb.astype(jnp.bfloat16).reshape(cb, H * W)
    a = xb.reshape(cb * H, W)
    r = jax.lax.broadcasted_iota(jnp.int32, (W, HP), 0)
    c = jax.lax.broadcasted_iota(jnp.int32, (W, HP), 1)
    p = jnp.where(r // POOL == c, 1.0 / (POOL * POOL), 0.0)
    aw = jnp.dot(a, p, preferred_element_type=jnp.float32)
    a3 = aw.reshape(cb, H, HP)
    rt = jax.lax.broadcasted_iota(jnp.int32, (HP, H), 0)
    ct = jax.lax.broadcasted_iota(jnp.int32, (HP, H), 1)
    pt = jnp.where(ct // POOL == rt, 1.0, 0.0)
    pooled = jnp.stack(
        [jnp.dot(pt, a3[ch], preferred_element_type=jnp.float32)
         for ch in range(cb)], axis=0)
    o_ref[...] = pooled.reshape(cb, NPP)


def _pool(x):
    x4 = x.reshape(B * CIN, H, W)
    cb = 16
    return pl.pallas_call(
        _pool_kernel,
        grid=(B * CIN // cb,),
        in_specs=[pl.BlockSpec((cb, H, W), lambda i: (i, 0, 0))],
        out_specs=[
            pl.BlockSpec((cb, NPP), lambda i: (i, 0)),
            pl.BlockSpec((cb, NPIX), lambda i: (i, 0)),
        ],
        out_shape=[
            jax.ShapeDtypeStruct((B * CIN, NPP), jnp.float32),
            jax.ShapeDtypeStruct((B * CIN, NPIX), jnp.bfloat16),
        ],
        compiler_params=pltpu.CompilerParams(
            dimension_semantics=("arbitrary",)),
    )(x4)


def _router_kernel(xd_ref, w1t_ref, c1_ref, w2_ref, c2_ref, ti_ref, tv_ref):
    xd = xd_ref[0]
    pad = HP + 1
    jcol = jax.lax.broadcasted_iota(jnp.int32, (1, NPP), 1) % HP
    acc = jnp.zeros((RED, NPP), jnp.float32)
    t = 0
    for di in (-1, 0, 1):
        for dj in (-1, 0, 1):
            s = HP * di + dj
            y = jnp.dot(w1t_ref[t], xd, preferred_element_type=jnp.float32)
            ys = jnp.pad(y, ((0, 0), (pad, pad)))[:, pad + s:pad + s + NPP]
            if dj == -1:
                ys = ys * (jcol >= 1).astype(jnp.float32)
            elif dj == 1:
                ys = ys * (jcol <= HP - 2).astype(jnp.float32)
            acc = acc + ys
            t += 1
    h = _silu(acc + c1_ref[...])
    m = jnp.sum(h, axis=1, keepdims=True) * (1.0 / NPP)
    gl = jnp.dot(w2_ref[...], m, preferred_element_type=jnp.float32) + c2_ref[...]
    ex = jnp.exp(gl - jnp.max(gl))
    prob = ex / jnp.sum(ex)
    ie = jax.lax.broadcasted_iota(jnp.int32, (E, 1), 0)
    v1 = jnp.max(prob)
    i1 = jnp.min(jnp.where(prob == v1, ie, E))
    p2 = jnp.where(ie == i1, -1.0, prob)
    v2 = jnp.max(p2)
    i2 = jnp.min(jnp.where(p2 == v2, ie, E))
    ssum = v1 + v2 + 1e-6
    iv = jax.lax.broadcasted_iota(jnp.int32, (1, 1, TOPK), 2)
    ti_ref[...] = jnp.where(iv == 0, i1, i2).astype(jnp.int32)
    tv_ref[...] = jnp.where(iv == 0, v1 / ssum, v2 / ssum)


def _router(xdf, w1t, c1, w2, c2):
    return pl.pallas_call(
        _router_kernel,
        grid=(B,),
        in_specs=[
            pl.BlockSpec((1, CIN, NPP), lambda b: (b, 0, 0)),
            pl.BlockSpec((9, RED, CIN), lambda b: (0, 0, 0)),
            pl.BlockSpec((RED, 1), lambda b: (0, 0)),
            pl.BlockSpec((E, RED), lambda b: (0, 0)),
            pl.BlockSpec((E, 1), lambda b: (0, 0)),
        ],
        out_specs=[
            pl.BlockSpec((1, 1, TOPK), lambda b: (b, 0, 0)),
            pl.BlockSpec((1, 1, TOPK), lambda b: (b, 0, 0)),
        ],
        out_shape=[
            jax.ShapeDtypeStruct((B, 1, TOPK), jnp.int32),
            jax.ShapeDtypeStruct((B, 1, TOPK), jnp.float32),
        ],
        compiler_params=pltpu.CompilerParams(
            dimension_semantics=("arbitrary",)),
    )(xdf, w1t, c1, w2, c2)


HT = 56
NT = HT * W


def _main_kernel(idx_ref, x_ref, tv_ref, w1a_ref, w1b_ref, c1a_ref, c1b_ref,
                 w2a_ref, w2b_ref, c2a_ref, c2b_ref, ws_ref, cs_ref, o_ref):
    w0 = tv_ref[0, 0, 0]
    w1 = tv_ref[0, 0, 1]
    wup = jnp.concatenate([w1a_ref[0], w1b_ref[0], ws_ref[...]], axis=0)
    bup = jnp.concatenate([c1a_ref[0], c1b_ref[0], cs_ref[...]], axis=0)
    wdn = jnp.concatenate([w0 * w2a_ref[0], w1 * w2b_ref[0]], axis=1)
    cc = w0 * c2a_ref[0] + w1 * c2b_ref[0]
    wup16 = wup.astype(jnp.bfloat16)
    wdn16 = wdn.astype(jnp.bfloat16)
    xt = x_ref[0]
    u = (jnp.dot(wup16, xt, preferred_element_type=jnp.float32)
         + bup).astype(jnp.bfloat16)
    v = _silu(u)
    res = (v[2 * HID:].astype(jnp.float32) + cc +
           jnp.dot(wdn16, v[:2 * HID],
                   preferred_element_type=jnp.float32))

    for r in range(HT):
        o_ref[0, :, r, :] = res[:, r * W:(r + 1) * W]


def _main(xf16, tif, tv, w1f, c1f, w2f, c2f, wsf, csf):
    grid_spec = pltpu.PrefetchScalarGridSpec(
        num_scalar_prefetch=1,
        grid=(B, H // HT),
        in_specs=[
            pl.BlockSpec((1, CIN, NT), lambda b, j, idx: (b, 0, j)),
            pl.BlockSpec((1, 1, TOPK), lambda b, j, idx: (b, 0, 0)),
            pl.BlockSpec((1, HID, CIN), lambda b, j, idx: (idx[2 * b], 0, 0)),
            pl.BlockSpec((1, HID, CIN), lambda b, j, idx: (idx[2 * b + 1], 0, 0)),
            pl.BlockSpec((1, HID, 1), lambda b, j, idx: (idx[2 * b], 0, 0)),
            pl.BlockSpec((1, HID, 1), lambda b, j, idx: (idx[2 * b + 1], 0, 0)),
            pl.BlockSpec((1, COUT, HID), lambda b, j, idx: (idx[2 * b], 0, 0)),
            pl.BlockSpec((1, COUT, HID), lambda b, j, idx: (idx[2 * b + 1], 0, 0)),
            pl.BlockSpec((1, COUT, 1), lambda b, j, idx: (idx[2 * b], 0, 0)),
            pl.BlockSpec((1, COUT, 1), lambda b, j, idx: (idx[2 * b + 1], 0, 0)),
            pl.BlockSpec((COUT, CIN), lambda b, j, idx: (0, 0)),
            pl.BlockSpec((COUT, 1), lambda b, j, idx: (0, 0)),
        ],
        out_specs=pl.BlockSpec((1, COUT, HT, W), lambda b, j, idx: (b, 0, j, 0)),
    )
    return pl.pallas_call(
        _main_kernel,
        grid_spec=grid_spec,
        out_shape=jax.ShapeDtypeStruct((B, COUT, H, W), jnp.float32),
        compiler_params=pltpu.CompilerParams(
            dimension_semantics=("parallel", "arbitrary")),
    )(tif, xf16, tv, w1f, w1f, c1f, c1f, w2f, w2f, c2f, c2f, wsf, csf)


def _fold(w, g, bb, m, v):
    s = g / jnp.sqrt(v + 1e-5)
    return w * s[..., None], (bb - m * s)


def kernel(x, r_w1, r_bn1_w, r_bn1_b, r_bn1_m, r_bn1_v, r_w2, r_bn2_w,
           r_bn2_b, r_bn2_m, r_bn2_v, s_w, s_bn_w, s_bn_b, s_bn_m, s_bn_v,
           e_w1, e_bn1_w, e_bn1_b, e_bn1_m, e_bn1_v, e_w2, e_bn2_w, e_bn2_b,
           e_bn2_m, e_bn2_v):
    xd, xflat16 = _pool(x)
    xdf = xd.reshape(B, CIN, NPP)

    rs1 = r_bn1_w / jnp.sqrt(r_bn1_v + 1e-5)
    w1t = (jnp.transpose(r_w1 * rs1[:, None, None, None], (2, 3, 0, 1))
           .reshape(9, RED, CIN))
    c1 = (r_bn1_b - r_bn1_m * rs1).reshape(RED, 1)
    rs2 = r_bn2_w / jnp.sqrt(r_bn2_v + 1e-5)
    w2 = r_w2[:, :, 0, 0] * rs2[:, None]
    c2 = (r_bn2_b - r_bn2_m * rs2).reshape(E, 1)
    ti, tv = _router(xdf, w1t, c1, w2, c2)
    tif = ti.reshape(B * TOPK)

    w1f, c1v = _fold(e_w1[:, :, :, 0, 0], e_bn1_w, e_bn1_b, e_bn1_m, e_bn1_v)
    c1f = c1v[:, :, None]
    w2f, c2v = _fold(e_w2[:, :, :, 0, 0], e_bn2_w, e_bn2_b, e_bn2_m, e_bn2_v)
    c2f = c2v[:, :, None]
    wsf, csv = _fold(s_w[:, :, 0, 0], s_bn_w, s_bn_b, s_bn_m, s_bn_v)
    csf = csv[:, None]
    return _main(xflat16.reshape(B, CIN, NPIX), tif, tv, w1f, c1f, w2f, c2f,
                 wsf, csf)
